# Initial kernel scaffold; baseline (speedup 1.0000x reference)
import functools

import jax
import jax.numpy as jnp
from jax import lax
from jax.experimental import pallas as pl
from jax.experimental.pallas import tpu as pltpu
from jax.experimental.pallas import tpu_sc as plsc

B, S, D = 2, 2048, 1024
E, K = 8, 2
T = B * S
BT = 512
BM = 256
PM = ((K * T + E * (BM - 1)) // BM + 1) * BM
NB = PM // BM

_EPS = 1e-5


def _a_body(x_ref, g1_ref, b1_ref, wg_ref, bg_ref,
            y_ref, w1_ref, w2_ref, i1_ref, i2_ref):
    x = x_ref[...]
    m = jnp.mean(x, axis=-1, keepdims=True)
    v = jnp.mean((x - m) ** 2, axis=-1, keepdims=True)
    ln = (x - m) / jnp.sqrt(v + _EPS) * g1_ref[...][None, :] + b1_ref[...][None, :]
    y = x + ln
    y_ref[...] = y
    logits = jnp.dot(y, wg_ref[...], preferred_element_type=jnp.float32)
    logits = logits + bg_ref[...][None, :]
    eidx = lax.broadcasted_iota(jnp.int32, logits.shape, 1)
    m1 = jnp.max(logits, axis=-1, keepdims=True)
    i1 = jnp.min(jnp.where(logits == m1, eidx, E), axis=-1)
    masked = jnp.where(eidx == i1[:, None], -jnp.inf, logits)
    m2 = jnp.max(masked, axis=-1, keepdims=True)
    i2 = jnp.min(jnp.where(masked == m2, eidx, E), axis=-1)
    sumexp = jnp.sum(jnp.exp(logits - m1), axis=-1)
    w1_ref[...] = 1.0 / sumexp
    w2_ref[...] = jnp.exp(m2[:, 0] - m1[:, 0]) / sumexp
    i1_ref[...] = i1
    i2_ref[...] = i2


def _stage_a(x2d, g1, b1, Wg, bg):
    nblk = T // BT
    return pl.pallas_call(
        _a_body,
        grid=(nblk,),
        in_specs=[
            pl.BlockSpec((BT, D), lambda i: (i, 0)),
            pl.BlockSpec((D,), lambda i: (0,)),
            pl.BlockSpec((D,), lambda i: (0,)),
            pl.BlockSpec((D, E), lambda i: (0, 0)),
            pl.BlockSpec((E,), lambda i: (0,)),
        ],
        out_specs=[
            pl.BlockSpec((BT, D), lambda i: (i, 0)),
            pl.BlockSpec((BT,), lambda i: (i,)),
            pl.BlockSpec((BT,), lambda i: (i,)),
            pl.BlockSpec((BT,), lambda i: (i,)),
            pl.BlockSpec((BT,), lambda i: (i,)),
        ],
        out_shape=[
            jax.ShapeDtypeStruct((T, D), jnp.float32),
            jax.ShapeDtypeStruct((T,), jnp.float32),
            jax.ShapeDtypeStruct((T,), jnp.float32),
            jax.ShapeDtypeStruct((T,), jnp.int32),
            jax.ShapeDtypeStruct((T,), jnp.int32),
        ],
    )(x2d, g1, b1, Wg, bg)


def _routing(i1, i2):
    e_flat = jnp.concatenate([i1, i2])
    oh = (e_flat[:, None] == jnp.arange(E)[None, :]).astype(jnp.int32)
    csum = jnp.cumsum(oh, axis=0)
    rank = jnp.sum((csum - oh) * oh, axis=1)
    counts = csum[-1]
    padded = ((counts + BM - 1) // BM) * BM
    cum = jnp.cumsum(padded)
    starts = cum - padded
    slot = starts[e_flat] + rank
    row_token = jnp.zeros((PM,), jnp.int32).at[slot].set(
        jnp.arange(K * T, dtype=jnp.int32) % T)
    blocks = jnp.arange(NB, dtype=jnp.int32) * BM
    be = jnp.clip(jnp.searchsorted(cum, blocks, side='right'), 0, E - 1)
    bvalid = ((blocks < cum[-1]) &
              (blocks - starts[be] < counts[be])).astype(jnp.int32)
    return row_token, slot, be.astype(jnp.int32), bvalid


_CH = 64

def _make_sc_gather(nrows, d):
    info = plsc.get_sparse_core_info()
    nc, ns = info.num_cores, info.num_subcores
    nw = nc * ns
    assert nrows % (nw * _CH) == 0
    chunks_per_w = nrows // (nw * _CH)
    mesh = plsc.VectorSubcoreMesh(core_axis_name="c", subcore_axis_name="s")

    @functools.partial(
        pl.kernel, mesh=mesh,
        out_type=jax.ShapeDtypeStruct((nrows, d), jnp.float32),
        scratch_types=[
            pltpu.VMEM((chunks_per_w, _CH), jnp.int32),
            pltpu.VMEM((_CH, d), jnp.float32),
            pltpu.SemaphoreType.DMA,
        ],
    )
    def gather(table_hbm, idx_hbm, out_hbm, idx_v, rows_v, sem):
        wid = lax.axis_index("s") * nc + lax.axis_index("c")
        crow0 = wid * chunks_per_w
        pltpu.sync_copy(idx_hbm.at[pl.ds(crow0, chunks_per_w)], idx_v)
        for c in range(chunks_per_w):
            pltpu.async_copy(table_hbm.at[idx_v.at[c]], rows_v, sem).wait()
            pltpu.sync_copy(
                rows_v, out_hbm.at[pl.ds((crow0 + c) * _CH, _CH)])

    def run(table, idx):
        return gather(table, idx.reshape(nrows // _CH, _CH))
    return run


_gather_x = _make_sc_gather(PM, D)
_gather_o = _make_sc_gather(K * T, D)


def _c_body(be_ref, bv_ref, x_ref, w_ref, b_ref, o_ref):
    i = pl.program_id(0)

    @pl.when(bv_ref[i] == 1)
    def _():
        o_ref[...] = (
            jnp.dot(x_ref[...], w_ref[0], preferred_element_type=jnp.float32)
            + b_ref[...])


def _stage_c(x_sorted, We, be_bias, block_expert, block_valid):
    grid_spec = pltpu.PrefetchScalarGridSpec(
        num_scalar_prefetch=2,
        grid=(NB,),
        in_specs=[
            pl.BlockSpec((BM, D), lambda i, be, bv: (i, 0)),
            pl.BlockSpec((1, D, D), lambda i, be, bv: (be[i], 0, 0)),
            pl.BlockSpec((1, D), lambda i, be, bv: (be[i], 0)),
        ],
        out_specs=pl.BlockSpec((BM, D), lambda i, be, bv: (i, 0)),
    )
    return pl.pallas_call(
        _c_body,
        grid_spec=grid_spec,
        out_shape=jax.ShapeDtypeStruct((PM, D), jnp.float32),
    )(block_expert, block_valid, x_sorted, We, be_bias)


def _d_body(y_ref, g0_ref, g1r_ref, w1_ref, w2_ref, g2_ref, b2_ref, o_ref):
    moe = (w1_ref[...][:, None] * g0_ref[...]
           + w2_ref[...][:, None] * g1r_ref[...])
    m = jnp.mean(moe, axis=-1, keepdims=True)
    v = jnp.mean((moe - m) ** 2, axis=-1, keepdims=True)
    ln = (moe - m) / jnp.sqrt(v + _EPS) * g2_ref[...][None, :] + b2_ref[...][None, :]
    o_ref[...] = y_ref[...] + ln


def _stage_d(y, G, w1, w2, g2, b2):
    nblk = T // BT
    return pl.pallas_call(
        _d_body,
        grid=(nblk,),
        in_specs=[
            pl.BlockSpec((BT, D), lambda i: (i, 0)),
            pl.BlockSpec((BT, D), lambda i: (i, 0)),
            pl.BlockSpec((BT, D), lambda i: (i + T // BT, 0)),
            pl.BlockSpec((BT,), lambda i: (i,)),
            pl.BlockSpec((BT,), lambda i: (i,)),
            pl.BlockSpec((D,), lambda i: (0,)),
            pl.BlockSpec((D,), lambda i: (0,)),
        ],
        out_specs=pl.BlockSpec((BT, D), lambda i: (i, 0)),
        out_shape=jax.ShapeDtypeStruct((T, D), jnp.float32),
    )(y, G, G, w1, w2, g2, b2)


def kernel(x, g1, b1, g2, b2, Wg, bg, We, be):
    x2d = x.reshape(T, D)
    y, w1, w2, i1, i2 = _stage_a(x2d, g1, b1, Wg, bg)
    row_token, slot, block_expert, block_valid = _routing(i1, i2)
    x_sorted = _gather_x(y, row_token)
    o_sorted = _stage_c(x_sorted, We, be, block_expert, block_valid)
    G = _gather_o(o_sorted, slot)
    out = _stage_d(y, G, w1, w2, g2, b2)
    return out.reshape(B, S, D)


# speedup vs baseline: 1.1586x; 1.1586x over previous
import functools

import jax
import jax.numpy as jnp
from jax import lax
from jax.experimental import pallas as pl
from jax.experimental.pallas import tpu as pltpu
from jax.experimental.pallas import tpu_sc as plsc

B, S, D = 2, 2048, 1024
E, K = 8, 2
T = B * S
BT = 512
BM = 256
PM = ((K * T + E * (BM - 1)) // BM + 1) * BM
NB = PM // BM

_EPS = 1e-5


def _a_body(x_ref, g1_ref, b1_ref, wg_ref, bg_ref,
            y_ref, w1_ref, w2_ref, i1_ref, i2_ref):
    x = x_ref[...]
    m = jnp.mean(x, axis=-1, keepdims=True)
    v = jnp.mean((x - m) ** 2, axis=-1, keepdims=True)
    ln = (x - m) / jnp.sqrt(v + _EPS) * g1_ref[...][None, :] + b1_ref[...][None, :]
    y = x + ln
    y_ref[...] = y
    logits = jnp.dot(y, wg_ref[...], preferred_element_type=jnp.float32)
    logits = logits + bg_ref[...][None, :]
    eidx = lax.broadcasted_iota(jnp.int32, logits.shape, 1)
    m1 = jnp.max(logits, axis=-1, keepdims=True)
    i1 = jnp.min(jnp.where(logits == m1, eidx, E), axis=-1)
    masked = jnp.where(eidx == i1[:, None], -jnp.inf, logits)
    m2 = jnp.max(masked, axis=-1, keepdims=True)
    i2 = jnp.min(jnp.where(masked == m2, eidx, E), axis=-1)
    sumexp = jnp.sum(jnp.exp(logits - m1), axis=-1)
    w1_ref[...] = 1.0 / sumexp
    w2_ref[...] = jnp.exp(m2[:, 0] - m1[:, 0]) / sumexp
    i1_ref[...] = i1
    i2_ref[...] = i2


def _stage_a(x2d, g1, b1, Wg, bg):
    nblk = T // BT
    return pl.pallas_call(
        _a_body,
        grid=(nblk,),
        in_specs=[
            pl.BlockSpec((BT, D), lambda i: (i, 0)),
            pl.BlockSpec((D,), lambda i: (0,)),
            pl.BlockSpec((D,), lambda i: (0,)),
            pl.BlockSpec((D, E), lambda i: (0, 0)),
            pl.BlockSpec((E,), lambda i: (0,)),
        ],
        out_specs=[
            pl.BlockSpec((BT, D), lambda i: (i, 0)),
            pl.BlockSpec((BT,), lambda i: (i,)),
            pl.BlockSpec((BT,), lambda i: (i,)),
            pl.BlockSpec((BT,), lambda i: (i,)),
            pl.BlockSpec((BT,), lambda i: (i,)),
        ],
        out_shape=[
            jax.ShapeDtypeStruct((T, D), jnp.float32),
            jax.ShapeDtypeStruct((T,), jnp.float32),
            jax.ShapeDtypeStruct((T,), jnp.float32),
            jax.ShapeDtypeStruct((T,), jnp.int32),
            jax.ShapeDtypeStruct((T,), jnp.int32),
        ],
    )(x2d, g1, b1, Wg, bg)


def _routing(i1, i2):
    e_flat = jnp.concatenate([i1, i2])
    oh = (e_flat[:, None] == jnp.arange(E)[None, :]).astype(jnp.int32)
    csum = jnp.cumsum(oh, axis=0)
    rank = jnp.sum((csum - oh) * oh, axis=1)
    counts = csum[-1]
    padded = ((counts + BM - 1) // BM) * BM
    cum = jnp.cumsum(padded)
    starts = cum - padded
    slot = starts[e_flat] + rank
    row_token = jnp.zeros((PM,), jnp.int32).at[slot].set(
        jnp.arange(K * T, dtype=jnp.int32) % T)
    blocks = jnp.arange(NB, dtype=jnp.int32) * BM
    be = jnp.clip(jnp.searchsorted(cum, blocks, side='right'), 0, E - 1)
    bvalid = ((blocks < cum[-1]) &
              (blocks - starts[be] < counts[be])).astype(jnp.int32)
    return row_token, slot, be.astype(jnp.int32), bvalid


_CH = 64

@functools.lru_cache(maxsize=None)
def _make_sc_gather(nrows, d):
    nc, ns = 2, 16
    nw = nc * ns
    assert nrows % (nw * _CH) == 0
    chunks_per_w = nrows // (nw * _CH)
    mesh = plsc.VectorSubcoreMesh(
        core_axis_name="c", subcore_axis_name="s",
        num_cores=nc, num_subcores=ns)

    @functools.partial(
        pl.kernel, mesh=mesh,
        out_type=jax.ShapeDtypeStruct((nrows, d), jnp.float32),
        scratch_types=[
            pltpu.VMEM((chunks_per_w, _CH), jnp.int32),
            pltpu.VMEM((_CH, d), jnp.float32),
            pltpu.SemaphoreType.DMA,
        ],
    )
    def gather(table_hbm, idx_hbm, out_hbm, idx_v, rows_v, sem):
        wid = lax.axis_index("s") * nc + lax.axis_index("c")
        pltpu.sync_copy(idx_hbm.at[wid], idx_v)
        for c in range(chunks_per_w):
            pltpu.async_copy(table_hbm.at[idx_v.at[c]], rows_v, sem).wait()
            pltpu.sync_copy(
                rows_v, out_hbm.at[pl.ds((wid * chunks_per_w + c) * _CH, _CH)])

    def run(table, idx):
        return gather(table, idx.reshape(nw, chunks_per_w, _CH))
    return run


def _gather_x(table, idx):
    return _make_sc_gather(PM, D)(table, idx)


def _gather_o(table, idx):
    return _make_sc_gather(K * T, D)(table, idx)


def _c_body(be_ref, bv_ref, x_ref, w_ref, b_ref, o_ref):
    i = pl.program_id(0)

    @pl.when(bv_ref[i] == 1)
    def _():
        o_ref[...] = (
            jnp.dot(x_ref[...], w_ref[0], preferred_element_type=jnp.float32)
            + b_ref[0])


def _stage_c(x_sorted, We, be_bias, block_expert, block_valid):
    grid_spec = pltpu.PrefetchScalarGridSpec(
        num_scalar_prefetch=2,
        grid=(NB,),
        in_specs=[
            pl.BlockSpec((BM, D), lambda i, be, bv: (i, 0)),
            pl.BlockSpec((1, D, D), lambda i, be, bv: (be[i], 0, 0)),
            pl.BlockSpec((1, 1, D), lambda i, be, bv: (be[i], 0, 0)),
        ],
        out_specs=pl.BlockSpec((BM, D), lambda i, be, bv: (i, 0)),
    )
    return pl.pallas_call(
        _c_body,
        grid_spec=grid_spec,
        out_shape=jax.ShapeDtypeStruct((PM, D), jnp.float32),
    )(block_expert, block_valid, x_sorted, We, be_bias.reshape(E, 1, D))


def _d_body(y_ref, g0_ref, g1r_ref, w1_ref, w2_ref, g2_ref, b2_ref, o_ref):
    moe = (w1_ref[...][:, None] * g0_ref[...]
           + w2_ref[...][:, None] * g1r_ref[...])
    m = jnp.mean(moe, axis=-1, keepdims=True)
    v = jnp.mean((moe - m) ** 2, axis=-1, keepdims=True)
    ln = (moe - m) / jnp.sqrt(v + _EPS) * g2_ref[...][None, :] + b2_ref[...][None, :]
    o_ref[...] = y_ref[...] + ln


def _stage_d(y, G, w1, w2, g2, b2):
    nblk = T // BT
    return pl.pallas_call(
        _d_body,
        grid=(nblk,),
        in_specs=[
            pl.BlockSpec((BT, D), lambda i: (i, 0)),
            pl.BlockSpec((BT, D), lambda i: (i, 0)),
            pl.BlockSpec((BT, D), lambda i: (i + T // BT, 0)),
            pl.BlockSpec((BT,), lambda i: (i,)),
            pl.BlockSpec((BT,), lambda i: (i,)),
            pl.BlockSpec((D,), lambda i: (0,)),
            pl.BlockSpec((D,), lambda i: (0,)),
        ],
        out_specs=pl.BlockSpec((BT, D), lambda i: (i, 0)),
        out_shape=jax.ShapeDtypeStruct((T, D), jnp.float32),
    )(y, G, G, w1, w2, g2, b2)


def kernel(x, g1, b1, g2, b2, Wg, bg, We, be):
    x2d = x.reshape(T, D)
    y, w1, w2, i1, i2 = _stage_a(x2d, g1, b1, Wg, bg)
    row_token, slot, block_expert, block_valid = _routing(i1, i2)
    x_sorted = _gather_x(y, row_token)
    o_sorted = _stage_c(x_sorted, We, be, block_expert, block_valid)
    G = _gather_o(o_sorted, slot)
    out = _stage_d(y, G, w1, w2, g2, b2)
    return out.reshape(B, S, D)
